# Initial kernel scaffold; baseline (speedup 1.0000x reference)
import jax
import jax.numpy as jnp
from jax.experimental import pallas as pl


def kernel(x, edge_index, W_emb, b_emb, Wp, Vp, bp, Wq, Vq, bq, gamma, beta, W1, b1, W2, b2, W3, b3):
    raise NotImplementedError("write your pallas kernel here")


# speedup vs baseline: 2.2270x; 2.2270x over previous
import functools

import jax
import jax.numpy as jnp
from jax import lax
from jax.experimental import pallas as pl
from jax.experimental.pallas import tpu as pltpu
from jax.experimental.pallas import tpu_sc as plsc

N = 10000
E = 320000
D_IN = 128
H = 64
NHID = 128
OUT = 10
NUM_LAYERS = 10
EPS = 0.1

_NC = 2
_NS = 16
_NW = _NC * _NS
_IB = 128
_EPW = 10240
_E_PAD = _EPW * _NW
_RPW = _EPW // _IB
_ACC_ROWS = 10112


def _sc_agg(gidx2, sidx2, m, zeros):
    mesh = plsc.VectorSubcoreMesh(
        core_axis_name="c", subcore_axis_name="s", num_cores=_NC,
        num_subcores=_NS)

    @functools.partial(
        pl.kernel,
        out_type=jax.ShapeDtypeStruct((_NC, _ACC_ROWS, NHID), jnp.float32),
        mesh=mesh,
        scratch_types=[
            pltpu.VMEM((_RPW, _IB), jnp.int32),
            pltpu.VMEM((_RPW, _IB), jnp.int32),
            pltpu.VMEM((_IB, NHID), jnp.float32),
            pltpu.VMEM_SHARED((_ACC_ROWS, NHID), jnp.float32),
            pltpu.SemaphoreType.DMA,
        ],
    )
    def k(gidx_hbm, sidx_hbm, m_hbm, z_hbm, out_hbm, gi, si,
          b0, acc_sh, sem0):
        c = lax.axis_index("c")
        s = lax.axis_index("s")
        w = s * _NC + c

        zr = _ACC_ROWS // _NS
        pltpu.sync_copy(z_hbm.at[pl.ds(s * zr, zr)],
                        acc_sh.at[pl.ds(s * zr, zr)])
        row0 = w * _RPW
        pltpu.sync_copy(gidx_hbm.at[pl.ds(row0, _RPW)], gi)
        pltpu.sync_copy(sidx_hbm.at[pl.ds(row0, _RPW)], si)
        plsc.subcore_barrier()

        def step(i, carry):
            pltpu.async_copy(m_hbm.at[gi.at[i]], b0, sem0).wait()
            pltpu.sync_copy(b0, acc_sh.at[si.at[i]], add=True)
            return carry

        lax.fori_loop(0, _RPW, step, 0)

        plsc.subcore_barrier()
        pltpu.sync_copy(acc_sh.at[pl.ds(s * zr, zr)],
                        out_hbm.at[c].at[pl.ds(s * zr, zr)])

    return k(gidx2, sidx2, m, zeros)


_BLK = 1000


def _gelu(v):
    return 0.5 * v * (1.0 + lax.erf(v * (2.0 ** -0.5)))


def _embed_call(x, wembT, b, vpT, vqT):
    def body(x_ref, w_ref, b_ref, vp_ref, vq_ref, o_ref, m_ref):
        acc = jnp.dot(x_ref[...], w_ref[...],
                      preferred_element_type=jnp.float32)
        g = _gelu(acc + b_ref[...])
        o_ref[...] = jnp.concatenate([g, g], axis=1)
        mp = jnp.dot(g, vp_ref[...], preferred_element_type=jnp.float32)
        mq = jnp.dot(g, vq_ref[...], preferred_element_type=jnp.float32)
        m_ref[...] = jnp.concatenate([mp, mq], axis=1)

    return pl.pallas_call(
        body,
        grid=(N // _BLK,),
        in_specs=[
            pl.BlockSpec((_BLK, D_IN), lambda i: (i, 0)),
            pl.BlockSpec((D_IN, H), lambda i: (0, 0)),
            pl.BlockSpec((1, H), lambda i: (0, 0)),
            pl.BlockSpec((H, H), lambda i: (0, 0)),
            pl.BlockSpec((H, H), lambda i: (0, 0)),
        ],
        out_specs=(pl.BlockSpec((_BLK, NHID), lambda i: (i, 0)),
                   pl.BlockSpec((_BLK, NHID), lambda i: (i, 0))),
        out_shape=(jax.ShapeDtypeStruct((N, NHID), jnp.float32),
                   jax.ShapeDtypeStruct((N, NHID), jnp.float32)),
    )(x, wembT, b, vpT, vqT)


def _stepA_call(h, part, wT, b, src_half):
    cs = src_half * H

    def body(h_ref, p0_ref, p1_ref, w_ref, b_ref, o_ref):
        xq = h_ref[:, cs:cs + H]
        u = p0_ref[0][:, cs:cs + H] + p1_ref[0][:, cs:cs + H]
        a = jnp.tanh(
            jnp.dot(xq, w_ref[...], preferred_element_type=jnp.float32)
            + u + b_ref[...])
        o_ref[...] = jnp.concatenate(
            [a, jnp.zeros((_BLK, H), jnp.float32)], axis=1)

    return pl.pallas_call(
        body,
        grid=(N // _BLK,),
        in_specs=[
            pl.BlockSpec((_BLK, NHID), lambda i: (i, 0)),
            pl.BlockSpec((1, _BLK, NHID), lambda i: (0, i, 0)),
            pl.BlockSpec((1, _BLK, NHID), lambda i: (1, i, 0)),
            pl.BlockSpec((H, H), lambda i: (0, 0)),
            pl.BlockSpec((1, H), lambda i: (0, 0)),
        ],
        out_specs=pl.BlockSpec((_BLK, NHID), lambda i: (i, 0)),
        out_shape=jax.ShapeDtypeStruct((N, NHID), jnp.float32),
    )(h, part, part, wT, b)


def _stepB_call(h, m, a, part, w, v, vprojT, scale, upd_half):
    cu = upd_half * H

    def body(h_ref, m_ref, a_ref, p0_ref, p1_ref, w_ref, v_ref, vp_ref,
             o_ref, mo_ref):
        hh = h_ref[...]
        mm = m_ref[...]
        aa = a_ref[:, :H]
        u = p0_ref[0][:, :H] + p1_ref[0][:, :H]
        upd = hh[:, cu:cu + H] + scale * (
            jnp.dot(aa, w_ref[...], preferred_element_type=jnp.float32)
            + jnp.dot(u, v_ref[...], preferred_element_type=jnp.float32))
        mupd = jnp.dot(upd, vp_ref[...], preferred_element_type=jnp.float32)
        if upd_half == 0:
            o_ref[...] = jnp.concatenate([upd, hh[:, H:]], axis=1)
            mo_ref[...] = jnp.concatenate([mupd, mm[:, H:]], axis=1)
        else:
            o_ref[...] = jnp.concatenate([hh[:, :H], upd], axis=1)
            mo_ref[...] = jnp.concatenate([mm[:, :H], mupd], axis=1)

    return pl.pallas_call(
        body,
        grid=(N // _BLK,),
        in_specs=[
            pl.BlockSpec((_BLK, NHID), lambda i: (i, 0)),
            pl.BlockSpec((_BLK, NHID), lambda i: (i, 0)),
            pl.BlockSpec((_BLK, NHID), lambda i: (i, 0)),
            pl.BlockSpec((1, _BLK, NHID), lambda i: (0, i, 0)),
            pl.BlockSpec((1, _BLK, NHID), lambda i: (1, i, 0)),
            pl.BlockSpec((H, H), lambda i: (0, 0)),
            pl.BlockSpec((H, H), lambda i: (0, 0)),
            pl.BlockSpec((H, H), lambda i: (0, 0)),
        ],
        out_specs=(pl.BlockSpec((_BLK, NHID), lambda i: (i, 0)),
                   pl.BlockSpec((_BLK, NHID), lambda i: (i, 0))),
        out_shape=(jax.ShapeDtypeStruct((N, NHID), jnp.float32),
                   jax.ShapeDtypeStruct((N, NHID), jnp.float32)),
    )(h, m, a, part, part, w, v, vprojT)


def _readout_call(h, gamma, beta, w1T, b1, w2T, b2, w3T, b3):
    def body(h_ref, g_ref, be_ref, w1_ref, b1_ref, w2_ref, b2_ref,
             w3_ref, b3_ref, o_ref):
        hh = h_ref[...]
        mu = jnp.mean(hh, axis=1, keepdims=True)
        var = jnp.mean((hh - mu) ** 2, axis=1, keepdims=True)
        hn = (hh - mu) / jnp.sqrt(var + 1e-5) * g_ref[...] + be_ref[...]
        h1 = _gelu(jnp.dot(hn, w1_ref[...],
                           preferred_element_type=jnp.float32) + b1_ref[...])
        h2 = _gelu(jnp.dot(h1, w2_ref[...],
                           preferred_element_type=jnp.float32) + b2_ref[...])
        o_ref[...] = jnp.dot(h2, w3_ref[...],
                             preferred_element_type=jnp.float32) + b3_ref[...]

    return pl.pallas_call(
        body,
        grid=(N // _BLK,),
        in_specs=[
            pl.BlockSpec((_BLK, NHID), lambda i: (i, 0)),
            pl.BlockSpec((1, NHID), lambda i: (0, 0)),
            pl.BlockSpec((1, NHID), lambda i: (0, 0)),
            pl.BlockSpec((NHID, NHID // 2), lambda i: (0, 0)),
            pl.BlockSpec((1, NHID // 2), lambda i: (0, 0)),
            pl.BlockSpec((NHID // 2, NHID // 2), lambda i: (0, 0)),
            pl.BlockSpec((1, NHID // 2), lambda i: (0, 0)),
            pl.BlockSpec((NHID // 2, OUT), lambda i: (0, 0)),
            pl.BlockSpec((1, OUT), lambda i: (0, 0)),
        ],
        out_specs=pl.BlockSpec((_BLK, OUT), lambda i: (i, 0)),
        out_shape=jax.ShapeDtypeStruct((N, OUT), jnp.float32),
    )(h, gamma, beta, w1T, b1, w2T, b2, w3T, b3)


def kernel(x, edge_index, W_emb, b_emb, Wp, Vp, bp, Wq, Vq, bq,
           gamma, beta, W1, b1, W2, b2, W3, b3):
    src, dst = edge_index[0], edge_index[1]
    pad = _E_PAD - E
    zpad = jnp.zeros((pad,), jnp.int32)
    dpad = jnp.full((pad,), N, jnp.int32)
    src_g = jnp.concatenate([src, zpad]).reshape(-1, _IB)
    src_s = jnp.concatenate([src, dpad]).reshape(-1, _IB)
    dst_g = jnp.concatenate([dst, zpad]).reshape(-1, _IB)
    dst_s = jnp.concatenate([dst, dpad]).reshape(-1, _IB)
    zeros = jnp.zeros((_ACC_ROWS, NHID), jnp.float32)

    def agg(m):
        return _sc_agg(src_g, dst_s, m, zeros)

    def agg_t(m):
        return _sc_agg(dst_g, src_s, m, zeros)

    bp2, bq2 = bp[None], bq[None]
    WpT, VpT = Wp.T, Vp.T
    WqT, VqT = Wq.T, Vq.T

    h, m = _embed_call(x, W_emb.T, b_emb[None], VpT, VqT)
    for _ in range(NUM_LAYERS):
        aq = _stepA_call(h, agg(m), WqT, bq2, src_half=1)
        h, m = _stepB_call(h, m, aq, agg_t(aq), Wq, Vq, VpT, -EPS,
                           upd_half=0)
        ap = _stepA_call(h, agg(m), WpT, bp2, src_half=0)
        h, m = _stepB_call(h, m, ap, agg_t(ap), Wp, Vp, VqT, EPS,
                           upd_half=1)

    return _readout_call(h, gamma[None], beta[None], W1.T, b1[None],
                         W2.T, b2[None], W3.T, b3[None])
